# baseline (speedup 1.0000x reference)
import jax
import jax.numpy as jnp
from jax.experimental import pallas as pl
from jax.experimental.pallas import tpu as pltpu

_D = 120
_NBINS = 121
_POOL = 8


def _prep_kernel(logits_ref, dm_ref, probs_ref, pooled_ref):
    x = logits_ref[0]
    m = jnp.max(x, axis=0, keepdims=True)
    e = jnp.exp(x - m)
    s = jnp.sum(e, axis=0, keepdims=True)
    probs_ref[0] = e[:_D] / s

    dm = dm_ref[0]
    hs, ws = dm.shape
    h, w = hs // _POOL, ws // _POOL
    ra = jax.lax.broadcasted_iota(jnp.int32, (h, hs), 0)
    ca = jax.lax.broadcasted_iota(jnp.int32, (h, hs), 1)
    pool_l = (ca // _POOL == ra).astype(jnp.float32)
    rb = jax.lax.broadcasted_iota(jnp.int32, (ws, w), 0)
    cb = jax.lax.broadcasted_iota(jnp.int32, (ws, w), 1)
    pool_r = (rb // _POOL == cb).astype(jnp.float32)
    hp = jax.lax.Precision.HIGHEST
    val = jnp.dot(
        jnp.dot(pool_l, dm, precision=hp, preferred_element_type=jnp.float32),
        pool_r, precision=hp, preferred_element_type=jnp.float32)
    nz = (dm != 0.0).astype(jnp.float32)
    cnt = jnp.dot(
        jnp.dot(pool_l, nz, precision=hp, preferred_element_type=jnp.float32),
        pool_r, precision=hp, preferred_element_type=jnp.float32)
    inv = 1.0 / (_POOL * _POOL)
    pooled_ref[0] = (val * inv) / (cnt * inv + 1e-10)


def _frustum_kernel(img_ref, probs_ref, bin_ref, out_ref, tgt_ref):
    iv = img_ref[0, 0]
    pv = probs_ref[0]
    out_ref[0, 0] = pv * iv[None]
    dd = jax.lax.broadcasted_iota(jnp.int32, pv.shape, 0)
    mask = dd == bin_ref[0][None]
    tgt_ref[0, 0] = jnp.where(mask, iv[None], 0.0)


def kernel(image_features, depth_logits, depth_maps, depth_target_bin):
    B, C, H, W = image_features.shape

    probs, pooled = pl.pallas_call(
        _prep_kernel,
        grid=(B,),
        in_specs=[
            pl.BlockSpec((1, _NBINS, H, W), lambda b: (b, 0, 0, 0)),
            pl.BlockSpec((1, H * _POOL, W * _POOL), lambda b: (b, 0, 0)),
        ],
        out_specs=[
            pl.BlockSpec((1, _D, H, W), lambda b: (b, 0, 0, 0)),
            pl.BlockSpec((1, H, W), lambda b: (b, 0, 0)),
        ],
        out_shape=[
            jax.ShapeDtypeStruct((B, _D, H, W), jnp.float32),
            jax.ShapeDtypeStruct((B, H, W), jnp.float32),
        ],
        compiler_params=pltpu.CompilerParams(
            dimension_semantics=("parallel",),
            vmem_limit_bytes=56 * 1024 * 1024,
        ),
        name="depth_ffn_prep",
    )(depth_logits, depth_maps)

    frustum, frustum_tgt = pl.pallas_call(
        _frustum_kernel,
        grid=(B, C),
        in_specs=[
            pl.BlockSpec((1, 1, H, W), lambda b, c: (b, c, 0, 0)),
            pl.BlockSpec((1, _D, H, W), lambda b, c: (b, 0, 0, 0)),
            pl.BlockSpec((1, H, W), lambda b, c: (b, 0, 0)),
        ],
        out_specs=[
            pl.BlockSpec((1, 1, _D, H, W), lambda b, c: (b, c, 0, 0, 0)),
            pl.BlockSpec((1, 1, _D, H, W), lambda b, c: (b, c, 0, 0, 0)),
        ],
        out_shape=[
            jax.ShapeDtypeStruct((B, C, _D, H, W), jnp.float32),
            jax.ShapeDtypeStruct((B, C, _D, H, W), jnp.float32),
        ],
        compiler_params=pltpu.CompilerParams(
            dimension_semantics=("parallel", "arbitrary"),
            vmem_limit_bytes=56 * 1024 * 1024,
        ),
        name="depth_ffn_frustum",
    )(image_features, probs, depth_target_bin)

    return frustum, frustum_tgt, pooled


# speedup vs baseline: 3.4475x; 3.4475x over previous
import jax
import jax.numpy as jnp
from jax.experimental import pallas as pl
from jax.experimental.pallas import tpu as pltpu

_D = 120
_NBINS = 121
_POOL = 8
_BLK = 512


def _prep_kernel(logits_ref, dm_ref, probs_ref, pooled_ref):
    x = logits_ref[0]
    m = jnp.max(x, axis=-1, keepdims=True)
    e = jnp.exp(x - m)
    s = jnp.sum(e, axis=-1, keepdims=True)
    probs_ref[0] = (e / s)[:, :_D]

    dm = dm_ref[0]
    hs, ws = dm.shape
    h, w = hs // _POOL, ws // _POOL
    ra = jax.lax.broadcasted_iota(jnp.int32, (h, hs), 0)
    ca = jax.lax.broadcasted_iota(jnp.int32, (h, hs), 1)
    pool_l = (ca // _POOL == ra).astype(jnp.float32)
    rb = jax.lax.broadcasted_iota(jnp.int32, (ws, w), 0)
    cb = jax.lax.broadcasted_iota(jnp.int32, (ws, w), 1)
    pool_r = (rb // _POOL == cb).astype(jnp.float32)
    hp = jax.lax.Precision.HIGHEST
    val = jnp.dot(
        jnp.dot(pool_l, dm, precision=hp, preferred_element_type=jnp.float32),
        pool_r, precision=hp, preferred_element_type=jnp.float32)
    nz = (dm != 0.0).astype(jnp.float32)
    cnt = jnp.dot(
        jnp.dot(pool_l, nz, precision=hp, preferred_element_type=jnp.float32),
        pool_r, precision=hp, preferred_element_type=jnp.float32)
    inv = 1.0 / (_POOL * _POOL)
    pooled_ref[0] = (val * inv) / (cnt * inv + 1e-10)


def _frustum_kernel(img_ref, probs_ref, bin_ref, out_ref, tgt_ref):
    img = img_ref[0]
    pv = probs_ref[0]
    bv = bin_ref[0, 0]
    p, c = img.shape
    img_b = jax.lax.broadcast_in_dim(img, (p, c, _D), (0, 1))
    pv_b = jax.lax.broadcast_in_dim(pv, (p, c, _D), (0, 2))
    out_ref[0] = img_b * pv_b
    dd = jax.lax.broadcasted_iota(jnp.int32, (p, _D), 1)
    mask = dd == bv
    mask_b = jax.lax.broadcast_in_dim(mask, (p, c, _D), (0, 2))
    tgt_ref[0] = jnp.where(mask_b, img_b, 0.0)


def kernel(image_features, depth_logits, depth_maps, depth_target_bin):
    B, C, H, W = image_features.shape
    N = H * W
    nb = -(-N // _BLK)

    logits_t = depth_logits.reshape(B, _NBINS, N).transpose(0, 2, 1)
    probs_t, pooled = pl.pallas_call(
        _prep_kernel,
        grid=(B,),
        in_specs=[
            pl.BlockSpec((1, N, _NBINS), lambda b: (b, 0, 0)),
            pl.BlockSpec((1, H * _POOL, W * _POOL), lambda b: (b, 0, 0)),
        ],
        out_specs=[
            pl.BlockSpec((1, N, _D), lambda b: (b, 0, 0)),
            pl.BlockSpec((1, H, W), lambda b: (b, 0, 0)),
        ],
        out_shape=[
            jax.ShapeDtypeStruct((B, N, _D), jnp.float32),
            jax.ShapeDtypeStruct((B, H, W), jnp.float32),
        ],
        compiler_params=pltpu.CompilerParams(
            dimension_semantics=("parallel",),
            vmem_limit_bytes=56 * 1024 * 1024,
        ),
        name="depth_ffn_prep",
    )(logits_t, depth_maps)

    img_t = image_features.reshape(B, C, N).transpose(0, 2, 1)
    bin_p = jnp.pad(depth_target_bin.reshape(B, N), ((0, 0), (0, nb * _BLK - N)))
    bin_p = bin_p.reshape(B, nb, _BLK, 1)

    out_t, tgt_t = pl.pallas_call(
        _frustum_kernel,
        grid=(B, nb),
        in_specs=[
            pl.BlockSpec((1, _BLK, C), lambda b, n: (b, n, 0)),
            pl.BlockSpec((1, _BLK, _D), lambda b, n: (b, n, 0)),
            pl.BlockSpec((1, 1, _BLK, 1), lambda b, n: (b, n, 0, 0)),
        ],
        out_specs=[
            pl.BlockSpec((1, _BLK, C, _D), lambda b, n: (b, n, 0, 0)),
            pl.BlockSpec((1, _BLK, C, _D), lambda b, n: (b, n, 0, 0)),
        ],
        out_shape=[
            jax.ShapeDtypeStruct((B, N, C, _D), jnp.float32),
            jax.ShapeDtypeStruct((B, N, C, _D), jnp.float32),
        ],
        compiler_params=pltpu.CompilerParams(
            dimension_semantics=("parallel", "arbitrary"),
            vmem_limit_bytes=56 * 1024 * 1024,
        ),
        name="depth_ffn_frustum",
    )(img_t, probs_t, bin_p)

    frustum = out_t.reshape(B, H, W, C, _D).transpose(0, 3, 4, 1, 2)
    frustum_tgt = tgt_t.reshape(B, H, W, C, _D).transpose(0, 3, 4, 1, 2)
    return frustum, frustum_tgt, pooled


# speedup vs baseline: 3.8364x; 1.1128x over previous
import jax
import jax.numpy as jnp
from jax.experimental import pallas as pl
from jax.experimental.pallas import tpu as pltpu

_D = 120
_NBINS = 121
_POOL = 8
_BLK = 512


def _prep_kernel(logits_ref, dm_ref, probs_ref, pooled_ref):
    x = logits_ref[0]
    m = jnp.max(x, axis=-1, keepdims=True)
    e = jnp.exp(x - m)
    s = jnp.sum(e, axis=-1, keepdims=True)
    probs_ref[0] = (e / s)[:, :_D]

    dm = dm_ref[0]
    ws, hs = dm.shape
    h, w = hs // _POOL, ws // _POOL
    ra = jax.lax.broadcasted_iota(jnp.int32, (w, ws), 0)
    ca = jax.lax.broadcasted_iota(jnp.int32, (w, ws), 1)
    pool_l = (ca // _POOL == ra).astype(jnp.float32)
    rb = jax.lax.broadcasted_iota(jnp.int32, (hs, h), 0)
    cb = jax.lax.broadcasted_iota(jnp.int32, (hs, h), 1)
    pool_r = (rb // _POOL == cb).astype(jnp.float32)
    hp = jax.lax.Precision.HIGHEST
    val = jnp.dot(
        jnp.dot(pool_l, dm, precision=hp, preferred_element_type=jnp.float32),
        pool_r, precision=hp, preferred_element_type=jnp.float32)
    nz = (dm != 0.0).astype(jnp.float32)
    cnt = jnp.dot(
        jnp.dot(pool_l, nz, preferred_element_type=jnp.float32),
        pool_r, preferred_element_type=jnp.float32)
    inv = 1.0 / (_POOL * _POOL)
    pooled_ref[0] = (val * inv) / (cnt * inv + 1e-10)


def _frustum_kernel(img_ref, probs_ref, bin_ref, out_ref, tgt_ref):
    img = img_ref[0]
    pv = probs_ref[0]
    bv = bin_ref[0]
    c, p = img.shape
    img_t = jnp.transpose(img)
    img_b = jax.lax.broadcast_in_dim(img_t, (p, c, _D), (0, 1))
    pv_b = jax.lax.broadcast_in_dim(pv, (p, c, _D), (0, 2))
    out_ref[0] = img_b * pv_b
    bv_t = jnp.transpose(bv)
    dd = jax.lax.broadcasted_iota(jnp.int32, (p, _D), 1)
    mask = dd == bv_t
    mask_b = jax.lax.broadcast_in_dim(mask, (p, c, _D), (0, 2))
    tgt_ref[0] = jnp.where(mask_b, img_b, 0.0)


def kernel(image_features, depth_logits, depth_maps, depth_target_bin):
    B, C, H, W = image_features.shape
    N = H * W
    nb = -(-N // _BLK)

    logits_t = depth_logits.reshape(B, _NBINS, N).transpose(0, 2, 1)
    dm_t = depth_maps.transpose(0, 2, 1)
    probs_t, pooled_t = pl.pallas_call(
        _prep_kernel,
        grid=(B,),
        in_specs=[
            pl.BlockSpec((1, N, _NBINS), lambda b: (b, 0, 0)),
            pl.BlockSpec((1, W * _POOL, H * _POOL), lambda b: (b, 0, 0)),
        ],
        out_specs=[
            pl.BlockSpec((1, N, _D), lambda b: (b, 0, 0)),
            pl.BlockSpec((1, W, H), lambda b: (b, 0, 0)),
        ],
        out_shape=[
            jax.ShapeDtypeStruct((B, N, _D), jnp.float32),
            jax.ShapeDtypeStruct((B, W, H), jnp.float32),
        ],
        compiler_params=pltpu.CompilerParams(
            dimension_semantics=("parallel",),
            vmem_limit_bytes=56 * 1024 * 1024,
        ),
        name="depth_ffn_prep",
    )(logits_t, dm_t)
    pooled = pooled_t.transpose(0, 2, 1)

    img_flat = image_features.reshape(B, C, N)
    bin_p = jnp.pad(depth_target_bin.reshape(B, N), ((0, 0), (0, nb * _BLK - N)))
    bin_p = bin_p.reshape(B * nb, 1, _BLK)

    out_t, tgt_t = pl.pallas_call(
        _frustum_kernel,
        grid=(B, nb),
        in_specs=[
            pl.BlockSpec((1, C, _BLK), lambda b, n: (b, 0, n)),
            pl.BlockSpec((1, _BLK, _D), lambda b, n: (b, n, 0)),
            pl.BlockSpec((1, 1, _BLK), lambda b, n: (b * nb + n, 0, 0)),
        ],
        out_specs=[
            pl.BlockSpec((1, _BLK, C, _D), lambda b, n: (b, n, 0, 0)),
            pl.BlockSpec((1, _BLK, C, _D), lambda b, n: (b, n, 0, 0)),
        ],
        out_shape=[
            jax.ShapeDtypeStruct((B, N, C, _D), jnp.float32),
            jax.ShapeDtypeStruct((B, N, C, _D), jnp.float32),
        ],
        compiler_params=pltpu.CompilerParams(
            dimension_semantics=("parallel", "arbitrary"),
            vmem_limit_bytes=56 * 1024 * 1024,
        ),
        name="depth_ffn_frustum",
    )(img_flat, probs_t, bin_p)

    frustum = out_t.reshape(B, H, W, C, _D).transpose(0, 3, 4, 1, 2)
    frustum_tgt = tgt_t.reshape(B, H, W, C, _D).transpose(0, 3, 4, 1, 2)
    return frustum, frustum_tgt, pooled


# speedup vs baseline: 3.9041x; 1.0177x over previous
import jax
import jax.numpy as jnp
from jax.experimental import pallas as pl
from jax.experimental.pallas import tpu as pltpu

_D = 120
_NBINS = 121
_POOL = 8
_BLK = 384


def _fused_kernel(logits_ref, dm_ref, img_ref, bin_ref,
                  out_ref, tgt_ref, pooled_ref, probs_s):
    n = pl.program_id(1)

    @pl.when(n == 0)
    def _prep():
        x = logits_ref[0]
        m = jnp.max(x, axis=-1, keepdims=True)
        e = jnp.exp(x - m)
        s = jnp.sum(e, axis=-1, keepdims=True)
        nn = x.shape[0]
        probs_s[pl.ds(0, nn), :] = (e / s)[:, :_D]

        dm = dm_ref[0]
        ws, hs = dm.shape
        h, w = hs // _POOL, ws // _POOL
        ra = jax.lax.broadcasted_iota(jnp.int32, (w, ws), 0)
        ca = jax.lax.broadcasted_iota(jnp.int32, (w, ws), 1)
        pool_l = (ca // _POOL == ra).astype(jnp.float32)
        rb = jax.lax.broadcasted_iota(jnp.int32, (hs, h), 0)
        cb = jax.lax.broadcasted_iota(jnp.int32, (hs, h), 1)
        pool_r = (rb // _POOL == cb).astype(jnp.float32)
        hp = jax.lax.Precision.HIGHEST
        val = jnp.dot(
            jnp.dot(pool_l, dm, precision=hp,
                    preferred_element_type=jnp.float32),
            pool_r, precision=hp, preferred_element_type=jnp.float32)
        nz = (dm != 0.0).astype(jnp.float32)
        cnt = jnp.dot(
            jnp.dot(pool_l, nz, preferred_element_type=jnp.float32),
            pool_r, preferred_element_type=jnp.float32)
        inv = 1.0 / (_POOL * _POOL)
        pooled_ref[0] = (val * inv) / (cnt * inv + 1e-10)

    img = img_ref[0]
    bv = bin_ref[0]
    c, p = img.shape
    pv = probs_s[pl.ds(n * _BLK, _BLK), :]
    img_t = jnp.transpose(img)
    img_b = jax.lax.broadcast_in_dim(img_t, (p, c, _D), (0, 1))
    pv_b = jax.lax.broadcast_in_dim(pv, (p, c, _D), (0, 2))
    out_ref[0] = img_b * pv_b
    bv_t = jnp.transpose(bv)
    dd = jax.lax.broadcasted_iota(jnp.int32, (p, _D), 1)
    mask = dd == bv_t
    mask_b = jax.lax.broadcast_in_dim(mask, (p, c, _D), (0, 2))
    tgt_ref[0] = jnp.where(mask_b, img_b, 0.0)


def kernel(image_features, depth_logits, depth_maps, depth_target_bin):
    B, C, H, W = image_features.shape
    N = H * W
    nb = -(-N // _BLK)

    logits_t = depth_logits.reshape(B, _NBINS, N).transpose(0, 2, 1)
    dm_t = depth_maps.transpose(0, 2, 1)
    img_flat = image_features.reshape(B, C, N)
    bin_p = jnp.pad(depth_target_bin.reshape(B, N), ((0, 0), (0, nb * _BLK - N)))
    bin_p = bin_p.reshape(B * nb, 1, _BLK)

    out_t, tgt_t, pooled_t = pl.pallas_call(
        _fused_kernel,
        grid=(B, nb),
        in_specs=[
            pl.BlockSpec((1, N, _NBINS), lambda b, n: (b, 0, 0)),
            pl.BlockSpec((1, W * _POOL, H * _POOL), lambda b, n: (b, 0, 0)),
            pl.BlockSpec((1, C, _BLK), lambda b, n: (b, 0, n)),
            pl.BlockSpec((1, 1, _BLK), lambda b, n: (b * nb + n, 0, 0)),
        ],
        out_specs=[
            pl.BlockSpec((1, _BLK, C, _D), lambda b, n: (b, n, 0, 0)),
            pl.BlockSpec((1, _BLK, C, _D), lambda b, n: (b, n, 0, 0)),
            pl.BlockSpec((1, W, H), lambda b, n: (b, 0, 0)),
        ],
        out_shape=[
            jax.ShapeDtypeStruct((B, N, C, _D), jnp.float32),
            jax.ShapeDtypeStruct((B, N, C, _D), jnp.float32),
            jax.ShapeDtypeStruct((B, W, H), jnp.float32),
        ],
        scratch_shapes=[pltpu.VMEM((nb * _BLK, _D), jnp.float32)],
        compiler_params=pltpu.CompilerParams(
            dimension_semantics=("parallel", "arbitrary"),
            vmem_limit_bytes=56 * 1024 * 1024,
        ),
        name="depth_ffn_fused",
    )(logits_t, dm_t, img_flat, bin_p)

    frustum = out_t.reshape(B, H, W, C, _D).transpose(0, 3, 4, 1, 2)
    frustum_tgt = tgt_t.reshape(B, H, W, C, _D).transpose(0, 3, 4, 1, 2)
    pooled = pooled_t.transpose(0, 2, 1)
    return frustum, frustum_tgt, pooled
